# speedup vs baseline: 1.6243x; 1.6243x over previous
import functools

import jax
import jax.numpy as jnp
from jax.experimental import pallas as pl
from jax.experimental.pallas import tpu as pltpu

T = 8192
D = 768
H = 256
E = 8
K = 2

BT = 512


def _moe_body(x_ref, W1_ref, b1_ref, W2_ref, b2_ref, W3_ref, b3_ref,
              G1_ref, g1_ref, G2_ref, g2_ref, G3_ref, g3_ref,
              P1_ref, p1_ref, P2_ref, p2_ref,
              out_ref, probs_ref, geo_ref):
    x = x_ref[...]

    gh = jax.nn.relu(jnp.dot(x, G1_ref[...], preferred_element_type=jnp.float32)
                     + g1_ref[...])
    gh = jax.nn.relu(jnp.dot(gh, G2_ref[...], preferred_element_type=jnp.float32)
                     + g2_ref[...])
    scores = jnp.dot(gh, G3_ref[...], preferred_element_type=jnp.float32) + g3_ref[...]
    m = jnp.max(scores, axis=1, keepdims=True)
    ex = jnp.exp(scores - m)
    probs = ex / jnp.sum(ex, axis=1, keepdims=True)
    probs_ref[...] = probs

    ids = jax.lax.broadcasted_iota(jnp.int32, (BT, E), 1)
    m1 = jnp.max(probs, axis=1, keepdims=True)
    i1 = jnp.min(jnp.where(probs == m1, ids, E), axis=1, keepdims=True)
    masked = jnp.where(ids == i1, -1.0, probs)
    m2 = jnp.max(masked, axis=1, keepdims=True)
    i2 = jnp.min(jnp.where(masked == m2, ids, E), axis=1, keepdims=True)
    den = m1 + m2
    w1 = m1 / den
    w2 = m2 / den

    ph = jax.nn.relu(jnp.dot(x, P1_ref[...], preferred_element_type=jnp.float32)
                     + p1_ref[...])
    geo_ref[...] = jnp.dot(ph, P2_ref[...], preferred_element_type=jnp.float32) \
        + p2_ref[...]

    acc = jnp.zeros((BT, D), dtype=jnp.float32)
    for e in range(E):
        h = jax.nn.relu(jnp.dot(x, W1_ref[e], preferred_element_type=jnp.float32)
                        + b1_ref[e][None, :])
        h = jax.nn.relu(jnp.dot(h, W2_ref[e], preferred_element_type=jnp.float32)
                        + b2_ref[e][None, :])
        o = jnp.dot(h, W3_ref[e], preferred_element_type=jnp.float32) \
            + b3_ref[e][None, :]
        coef = jnp.where(i1 == e, w1, 0.0) + jnp.where(i2 == e, w2, 0.0)
        acc = acc + coef * o
    out_ref[...] = acc


@jax.jit
def kernel(x, W1, b1, W2, b2, W3, b3, G1, g1, G2, g2, G3, g3, P1, p1, P2, p2):
    g1_ = g1.reshape(1, H)
    g2_ = g2.reshape(1, H)
    g3_ = g3.reshape(1, E)
    p1_ = p1.reshape(1, H)
    p2_ = p2.reshape(1, 1)

    full = lambda *shape: pl.BlockSpec(shape, lambda i, s=len(shape): (0,) * s)
    grid = (T // BT,)
    out, probs, geo = pl.pallas_call(
        _moe_body,
        grid=grid,
        in_specs=[
            pl.BlockSpec((BT, D), lambda i: (i, 0)),
            full(E, D, H), full(E, H), full(E, H, H), full(E, H),
            full(E, H, D), full(E, D),
            full(D, H), full(1, H), full(H, H), full(1, H),
            full(H, E), full(1, E),
            full(D, H), full(1, H), full(H, 1), full(1, 1),
        ],
        out_specs=[
            pl.BlockSpec((BT, D), lambda i: (i, 0)),
            pl.BlockSpec((BT, E), lambda i: (i, 0)),
            pl.BlockSpec((BT, 1), lambda i: (i, 0)),
        ],
        out_shape=[
            jax.ShapeDtypeStruct((T, D), jnp.float32),
            jax.ShapeDtypeStruct((T, E), jnp.float32),
            jax.ShapeDtypeStruct((T, 1), jnp.float32),
        ],
    )(x, W1, b1, W2, b2, W3, b3, G1, g1_, G2, g2_, G3, g3_, P1, p1_, P2, p2_)
    return out, probs, geo
